# baseline (speedup 1.0000x reference)
import jax
import jax.numpy as jnp
from jax import lax
from jax.experimental import pallas as pl
from jax.experimental.pallas import tpu as pltpu
from jax.experimental.pallas import tpu_sc as plsc

N = 10000
D = 128
E = 320000
L = 3

NC, NS = 2, 16
NW = NC * NS
EPW = E // NW
CHUNK = 128
NFULL = EPW // CHUNK
TAIL = EPW - NFULL * CHUNK
RPT = 624
RREM = N - NS * RPT

NB = 3
NI = 6
ROWS_BLK = 2000
GRID = N // ROWS_BLK


def _mm_body(x_ref, w_ref, o_ref):
    o_ref[...] = jnp.dot(x_ref[...], w_ref[...],
                         preferred_element_type=jnp.float32)


def _mm(x, w):
    return pl.pallas_call(
        _mm_body,
        grid=(GRID,),
        in_specs=[
            pl.BlockSpec((ROWS_BLK, D), lambda i: (i, 0)),
            pl.BlockSpec((D, D), lambda i: (0, 0)),
        ],
        out_specs=pl.BlockSpec((ROWS_BLK, D), lambda i: (i, 0)),
        out_shape=jax.ShapeDtypeStruct((N, D), jnp.float32),
    )(x, w)


def _gru_math(a0_ref, a1_ref, h_ref, wih_ref, whh_ref, bih_ref, bhh_ref):
    agg = a0_ref[...] + a1_ref[...]
    h = h_ref[...]
    gi = jnp.dot(agg, wih_ref[...], preferred_element_type=jnp.float32)
    gi = gi + bih_ref[...]
    gh = jnp.dot(h, whh_ref[...], preferred_element_type=jnp.float32)
    gh = gh + bhh_ref[...]
    r = jax.nn.sigmoid(gi[:, :D] + gh[:, :D])
    z = jax.nn.sigmoid(gi[:, D:2 * D] + gh[:, D:2 * D])
    n = jnp.tanh(gi[:, 2 * D:] + r * gh[:, 2 * D:])
    return (1.0 - z) * n + z * h


def _gru_body(a0_ref, a1_ref, h_ref, wih_ref, whh_ref, bih_ref, bhh_ref,
              o_ref):
    o_ref[...] = _gru_math(a0_ref, a1_ref, h_ref, wih_ref, whh_ref,
                           bih_ref, bhh_ref)


def _gru_mm_body(a0_ref, a1_ref, h_ref, wih_ref, whh_ref, bih_ref,
                 bhh_ref, wn_ref, o_ref, m_ref):
    hnew = _gru_math(a0_ref, a1_ref, h_ref, wih_ref, whh_ref, bih_ref,
                     bhh_ref)
    o_ref[...] = hnew
    m_ref[...] = jnp.dot(hnew, wn_ref[...],
                         preferred_element_type=jnp.float32)


_ROW_SPECS = [
    pl.BlockSpec((ROWS_BLK, D), lambda i: (i, 0)),
    pl.BlockSpec((ROWS_BLK, D), lambda i: (i, 0)),
    pl.BlockSpec((ROWS_BLK, D), lambda i: (i, 0)),
    pl.BlockSpec((D, 3 * D), lambda i: (0, 0)),
    pl.BlockSpec((D, 3 * D), lambda i: (0, 0)),
    pl.BlockSpec((1, 3 * D), lambda i: (0, 0)),
    pl.BlockSpec((1, 3 * D), lambda i: (0, 0)),
]


def _gru(a0, a1, h, wihT, whhT, bih, bhh):
    return pl.pallas_call(
        _gru_body,
        grid=(GRID,),
        in_specs=list(_ROW_SPECS),
        out_specs=pl.BlockSpec((ROWS_BLK, D), lambda i: (i, 0)),
        out_shape=jax.ShapeDtypeStruct((N, D), jnp.float32),
    )(a0, a1, h, wihT, whhT, bih, bhh)


def _gru_mm(a0, a1, h, wihT, whhT, bih, bhh, wnext):
    return pl.pallas_call(
        _gru_mm_body,
        grid=(GRID,),
        in_specs=list(_ROW_SPECS) + [pl.BlockSpec((D, D), lambda i: (0, 0))],
        out_specs=[
            pl.BlockSpec((ROWS_BLK, D), lambda i: (i, 0)),
            pl.BlockSpec((ROWS_BLK, D), lambda i: (i, 0)),
        ],
        out_shape=[
            jax.ShapeDtypeStruct((N, D), jnp.float32),
            jax.ShapeDtypeStruct((N, D), jnp.float32),
        ],
    )(a0, a1, h, wihT, whhT, bih, bhh, wnext)


def _sc_body(m_hbm, idx_hbm, idxt_hbm, zeros_hbm, out_hbm,
             agg_sh, idx0, idx1, idx2, idx3, idx4, idx5, idxt,
             rows0, rows1, rows2,
             isem0, isem1, isem2, isem3, isem4, isem5,
             gsem0, gsem1, gsem2, ssem0, ssem1, ssem2):
    cid = lax.axis_index("c")
    sid = lax.axis_index("s")
    w = cid * NS + sid

    pltpu.sync_copy(zeros_hbm.at[pl.ds(sid * RPT, RPT)],
                    agg_sh.at[pl.ds(sid * RPT, RPT)])

    @pl.when(sid == NS - 1)
    def _():
        pltpu.sync_copy(zeros_hbm.at[pl.ds(NS * RPT, RREM)],
                        agg_sh.at[pl.ds(NS * RPT, RREM)])

    plsc.subcore_barrier()

    idx = (idx0, idx1, idx2, idx3, idx4, idx5)
    isem = (isem0, isem1, isem2, isem3, isem4, isem5)
    rows = (rows0, rows1, rows2)
    gsem = (gsem0, gsem1, gsem2)
    ssem = (ssem0, ssem1, ssem2)

    def fire_idx(j, ib):
        pltpu.async_copy(idx_hbm.at[w, j], idx[ib], isem[ib])

    def wait_idx(ib):
        pltpu.make_async_copy(idx_hbm.at[w, 0], idx[ib], isem[ib]).wait()

    def fire_gather(j, ib, rb):
        pltpu.async_copy(m_hbm.at[idx[ib].at[0]], rows[rb], gsem[rb])

    def wait_gather(ib, rb):
        pltpu.make_async_copy(m_hbm.at[idx[ib].at[0]], rows[rb],
                              gsem[rb]).wait()

    def fire_scatter(ib, rb):
        pltpu.async_copy(rows[rb], agg_sh.at[idx[ib].at[1]], ssem[rb],
                         add=True)

    def wait_scatter(ib, rb):
        pltpu.make_async_copy(rows[rb], agg_sh.at[idx[ib].at[1]],
                              ssem[rb]).wait()

    for b in range(4):
        fire_idx(b, b)
    wait_idx(0)
    fire_gather(0, 0, 0)

    def step6(t, carry):
        j6 = 6 * t
        for b in range(6):
            j = j6 + b
            rb = b % NB
            wait_gather(b, rb)
            fire_scatter(b, rb)

            @pl.when(j >= 2)
            def _():
                wait_scatter((b - 2) % NI, (b - 2) % NB)

            @pl.when(j + 4 < NFULL)
            def _():
                fire_idx(j + 4, (b + 4) % NI)

            @pl.when(j + 1 < NFULL)
            def _():
                wait_idx((b + 1) % NI)
                fire_gather(j + 1, (b + 1) % NI, (b + 1) % NB)
        return carry

    lax.fori_loop(0, NFULL // 6, step6, 0)

    wait_scatter((NFULL - 2) % NI, (NFULL - 2) % NB)
    wait_scatter((NFULL - 1) % NI, (NFULL - 1) % NB)

    rows_t = rows0.at[pl.ds(0, TAIL)]
    pltpu.sync_copy(idxt_hbm.at[w], idxt)
    pltpu.async_copy(m_hbm.at[idxt.at[0]], rows_t, gsem0).wait()
    pltpu.sync_copy(rows_t, agg_sh.at[idxt.at[1]], add=True)

    plsc.subcore_barrier()
    pltpu.sync_copy(agg_sh.at[pl.ds(sid * RPT, RPT)],
                    out_hbm.at[cid, pl.ds(sid * RPT, RPT)])

    @pl.when(sid == NS - 1)
    def _():
        pltpu.sync_copy(agg_sh.at[pl.ds(NS * RPT, RREM)],
                        out_hbm.at[cid, pl.ds(NS * RPT, RREM)])


_SC_CACHE = {}


def _sc_segsum_call():
    if "k" not in _SC_CACHE:
        _SC_CACHE["k"] = pl.kernel(
            _sc_body,
            out_type=jax.ShapeDtypeStruct((NC, N, D), jnp.float32),
            mesh=plsc.VectorSubcoreMesh(core_axis_name="c",
                                        subcore_axis_name="s",
                                        num_cores=NC, num_subcores=NS),
            scratch_types=[
                pltpu.VMEM_SHARED((N, D), jnp.float32),
                pltpu.VMEM((2, CHUNK), jnp.int32),
                pltpu.VMEM((2, CHUNK), jnp.int32),
                pltpu.VMEM((2, CHUNK), jnp.int32),
                pltpu.VMEM((2, CHUNK), jnp.int32),
                pltpu.VMEM((2, CHUNK), jnp.int32),
                pltpu.VMEM((2, CHUNK), jnp.int32),
                pltpu.VMEM((2, TAIL), jnp.int32),
                pltpu.VMEM((CHUNK, D), jnp.float32),
                pltpu.VMEM((CHUNK, D), jnp.float32),
                pltpu.VMEM((CHUNK, D), jnp.float32),
                pltpu.SemaphoreType.DMA,
                pltpu.SemaphoreType.DMA,
                pltpu.SemaphoreType.DMA,
                pltpu.SemaphoreType.DMA,
                pltpu.SemaphoreType.DMA,
                pltpu.SemaphoreType.DMA,
                pltpu.SemaphoreType.DMA,
                pltpu.SemaphoreType.DMA,
                pltpu.SemaphoreType.DMA,
                pltpu.SemaphoreType.DMA,
                pltpu.SemaphoreType.DMA,
                pltpu.SemaphoreType.DMA,
            ],
        )
    return _SC_CACHE["k"]


def kernel(x_encoded, edge_index, mapping_idx, weight, w_ih, w_hh, b_ih,
           b_hh):
    del mapping_idx
    src = edge_index[0].reshape(NW, EPW)
    dst = edge_index[1].reshape(NW, EPW)
    idx_main = jnp.stack(
        [src[:, :NFULL * CHUNK].reshape(NW, NFULL, CHUNK),
         dst[:, :NFULL * CHUNK].reshape(NW, NFULL, CHUNK)], axis=2)
    idx_tail = jnp.stack([src[:, NFULL * CHUNK:], dst[:, NFULL * CHUNK:]],
                         axis=1)

    wihT = w_ih.T
    whhT = w_hh.T
    bih = b_ih.reshape(1, 3 * D)
    bhh = b_hh.reshape(1, 3 * D)
    zeros = jnp.zeros((N, D), jnp.float32)

    h = x_encoded
    m = _mm(h, weight[0])
    for i in range(L):
        parts = _sc_segsum_call()(m, idx_main, idx_tail, zeros)
        if i + 1 < L:
            h, m = _gru_mm(parts[0], parts[1], h, wihT, whhT, bih, bhh,
                           weight[i + 1])
        else:
            h = _gru(parts[0], parts[1], h, wihT, whhT, bih, bhh)
    return h


# speedup vs baseline: 1.1842x; 1.1842x over previous
import jax
import jax.numpy as jnp
from jax import lax
from jax.experimental import pallas as pl
from jax.experimental.pallas import tpu as pltpu
from jax.experimental.pallas import tpu_sc as plsc

N = 10000
D = 128
E = 320000
L = 3

NC, NS = 2, 16
NW = NC * NS
EPW = E // NW
CHUNK = 128
NFULL = EPW // CHUNK
TAIL = EPW - NFULL * CHUNK
RPT = 624
RREM = N - NS * RPT

NB = 3
NI = 6
ROWS_BLK = 2000
GRID = N // ROWS_BLK


def _mm_body(x_ref, w_ref, o_ref):
    o_ref[...] = jnp.dot(x_ref[...], w_ref[...],
                         preferred_element_type=jnp.float32)


def _mm(x, w):
    return pl.pallas_call(
        _mm_body,
        grid=(GRID,),
        in_specs=[
            pl.BlockSpec((ROWS_BLK, D), lambda i: (i, 0)),
            pl.BlockSpec((D, D), lambda i: (0, 0)),
        ],
        out_specs=pl.BlockSpec((ROWS_BLK, D), lambda i: (i, 0)),
        out_shape=jax.ShapeDtypeStruct((N, D), jnp.float32),
    )(x, w)


def _gru_math(a0_ref, a1_ref, h_ref, wih_ref, whh_ref, bih_ref, bhh_ref):
    agg = a0_ref[...] + a1_ref[...]
    h = h_ref[...]
    gi = jnp.dot(agg, wih_ref[...], preferred_element_type=jnp.float32)
    gi = gi + bih_ref[...]
    gh = jnp.dot(h, whh_ref[...], preferred_element_type=jnp.float32)
    gh = gh + bhh_ref[...]
    r = jax.nn.sigmoid(gi[:, :D] + gh[:, :D])
    z = jax.nn.sigmoid(gi[:, D:2 * D] + gh[:, D:2 * D])
    n = jnp.tanh(gi[:, 2 * D:] + r * gh[:, 2 * D:])
    return (1.0 - z) * n + z * h


def _gru_body(a0_ref, a1_ref, h_ref, wih_ref, whh_ref, bih_ref, bhh_ref,
              o_ref):
    o_ref[...] = _gru_math(a0_ref, a1_ref, h_ref, wih_ref, whh_ref,
                           bih_ref, bhh_ref)


def _gru_mm_body(a0_ref, a1_ref, h_ref, wih_ref, whh_ref, bih_ref,
                 bhh_ref, wn_ref, o_ref, m_ref):
    hnew = _gru_math(a0_ref, a1_ref, h_ref, wih_ref, whh_ref, bih_ref,
                     bhh_ref)
    o_ref[...] = hnew
    m_ref[...] = jnp.dot(hnew, wn_ref[...],
                         preferred_element_type=jnp.float32)


_ROW_SPECS = [
    pl.BlockSpec((ROWS_BLK, D), lambda i: (i, 0)),
    pl.BlockSpec((ROWS_BLK, D), lambda i: (i, 0)),
    pl.BlockSpec((ROWS_BLK, D), lambda i: (i, 0)),
    pl.BlockSpec((D, 3 * D), lambda i: (0, 0)),
    pl.BlockSpec((D, 3 * D), lambda i: (0, 0)),
    pl.BlockSpec((1, 3 * D), lambda i: (0, 0)),
    pl.BlockSpec((1, 3 * D), lambda i: (0, 0)),
]


def _gru(a0, a1, h, wihT, whhT, bih, bhh):
    return pl.pallas_call(
        _gru_body,
        grid=(GRID,),
        in_specs=list(_ROW_SPECS),
        out_specs=pl.BlockSpec((ROWS_BLK, D), lambda i: (i, 0)),
        out_shape=jax.ShapeDtypeStruct((N, D), jnp.float32),
    )(a0, a1, h, wihT, whhT, bih, bhh)


def _gru_mm(a0, a1, h, wihT, whhT, bih, bhh, wnext):
    return pl.pallas_call(
        _gru_mm_body,
        grid=(GRID,),
        in_specs=list(_ROW_SPECS) + [pl.BlockSpec((D, D), lambda i: (0, 0))],
        out_specs=[
            pl.BlockSpec((ROWS_BLK, D), lambda i: (i, 0)),
            pl.BlockSpec((ROWS_BLK, D), lambda i: (i, 0)),
        ],
        out_shape=[
            jax.ShapeDtypeStruct((N, D), jnp.float32),
            jax.ShapeDtypeStruct((N, D), jnp.float32),
        ],
    )(a0, a1, h, wihT, whhT, bih, bhh, wnext)


def _sc_body(m_hbm, idx_hbm, idxt_hbm, zeros_hbm, out_hbm,
             agg_sh, idx0, idx1, idx2, idx3, idx4, idx5, idxt,
             rows0, rows1, rows2,
             isem0, isem1, isem2, isem3, isem4, isem5,
             gsem0, gsem1, gsem2, ssem0, ssem1, ssem2):
    cid = lax.axis_index("c")
    sid = lax.axis_index("s")
    w = cid * NS + sid

    pltpu.sync_copy(zeros_hbm.at[pl.ds(sid * RPT, RPT)],
                    agg_sh.at[pl.ds(sid * RPT, RPT)])

    @pl.when(sid == NS - 1)
    def _():
        pltpu.sync_copy(zeros_hbm.at[pl.ds(NS * RPT, RREM)],
                        agg_sh.at[pl.ds(NS * RPT, RREM)])

    plsc.subcore_barrier()

    idx = (idx0, idx1, idx2, idx3, idx4, idx5)
    isem = (isem0, isem1, isem2, isem3, isem4, isem5)
    rows = (rows0, rows1, rows2)
    gsem = (gsem0, gsem1, gsem2)
    ssem = (ssem0, ssem1, ssem2)

    def fire_idx(j, ib):
        pltpu.async_copy(idx_hbm.at[w, j], idx[ib], isem[ib])

    def wait_idx(ib):
        pltpu.make_async_copy(idx_hbm.at[w, 0], idx[ib], isem[ib]).wait()

    def fire_gather(j, ib, rb):
        pltpu.async_copy(m_hbm.at[idx[ib].at[0]], rows[rb], gsem[rb])

    def wait_gather(ib, rb):
        pltpu.make_async_copy(m_hbm.at[idx[ib].at[0]], rows[rb],
                              gsem[rb]).wait()

    def fire_scatter(ib, rb):
        pltpu.async_copy(rows[rb], agg_sh.at[idx[ib].at[1]], ssem[rb],
                         add=True)

    def wait_scatter(ib, rb):
        pltpu.make_async_copy(rows[rb], agg_sh.at[idx[ib].at[1]],
                              ssem[rb]).wait()

    for b in range(4):
        fire_idx(b, b)
    wait_idx(0)
    fire_gather(0, 0, 0)
    wait_idx(1)
    fire_gather(1, 1, 1)

    def step6(t, carry):
        j6 = 6 * t
        for b in range(6):
            j = j6 + b
            rb = b % NB
            wait_gather(b, rb)
            fire_scatter(b, rb)

            @pl.when(j >= 1)
            def _():
                wait_scatter((b - 1) % NI, (b - 1) % NB)

            @pl.when(j + 4 < NFULL)
            def _():
                fire_idx(j + 4, (b + 4) % NI)

            @pl.when(j + 2 < NFULL)
            def _():
                wait_idx((b + 2) % NI)
                fire_gather(j + 2, (b + 2) % NI, (b + 2) % NB)
        return carry

    lax.fori_loop(0, NFULL // 6, step6, 0)

    wait_scatter((NFULL - 1) % NI, (NFULL - 1) % NB)

    rows_t = rows0.at[pl.ds(0, TAIL)]
    pltpu.sync_copy(idxt_hbm.at[w], idxt)
    pltpu.async_copy(m_hbm.at[idxt.at[0]], rows_t, gsem0).wait()
    pltpu.sync_copy(rows_t, agg_sh.at[idxt.at[1]], add=True)

    plsc.subcore_barrier()
    pltpu.sync_copy(agg_sh.at[pl.ds(sid * RPT, RPT)],
                    out_hbm.at[cid, pl.ds(sid * RPT, RPT)])

    @pl.when(sid == NS - 1)
    def _():
        pltpu.sync_copy(agg_sh.at[pl.ds(NS * RPT, RREM)],
                        out_hbm.at[cid, pl.ds(NS * RPT, RREM)])


_SC_CACHE = {}


def _sc_segsum_call():
    if "k" not in _SC_CACHE:
        _SC_CACHE["k"] = pl.kernel(
            _sc_body,
            out_type=jax.ShapeDtypeStruct((NC, N, D), jnp.float32),
            mesh=plsc.VectorSubcoreMesh(core_axis_name="c",
                                        subcore_axis_name="s",
                                        num_cores=NC, num_subcores=NS),
            scratch_types=[
                pltpu.VMEM_SHARED((N, D), jnp.float32),
                pltpu.VMEM((2, CHUNK), jnp.int32),
                pltpu.VMEM((2, CHUNK), jnp.int32),
                pltpu.VMEM((2, CHUNK), jnp.int32),
                pltpu.VMEM((2, CHUNK), jnp.int32),
                pltpu.VMEM((2, CHUNK), jnp.int32),
                pltpu.VMEM((2, CHUNK), jnp.int32),
                pltpu.VMEM((2, TAIL), jnp.int32),
                pltpu.VMEM((CHUNK, D), jnp.float32),
                pltpu.VMEM((CHUNK, D), jnp.float32),
                pltpu.VMEM((CHUNK, D), jnp.float32),
                pltpu.SemaphoreType.DMA,
                pltpu.SemaphoreType.DMA,
                pltpu.SemaphoreType.DMA,
                pltpu.SemaphoreType.DMA,
                pltpu.SemaphoreType.DMA,
                pltpu.SemaphoreType.DMA,
                pltpu.SemaphoreType.DMA,
                pltpu.SemaphoreType.DMA,
                pltpu.SemaphoreType.DMA,
                pltpu.SemaphoreType.DMA,
                pltpu.SemaphoreType.DMA,
                pltpu.SemaphoreType.DMA,
            ],
        )
    return _SC_CACHE["k"]


def kernel(x_encoded, edge_index, mapping_idx, weight, w_ih, w_hh, b_ih,
           b_hh):
    del mapping_idx
    src = edge_index[0].reshape(NW, EPW)
    dst = edge_index[1].reshape(NW, EPW)
    idx_main = jnp.stack(
        [src[:, :NFULL * CHUNK].reshape(NW, NFULL, CHUNK),
         dst[:, :NFULL * CHUNK].reshape(NW, NFULL, CHUNK)], axis=2)
    idx_tail = jnp.stack([src[:, NFULL * CHUNK:], dst[:, NFULL * CHUNK:]],
                         axis=1)

    wihT = w_ih.T
    whhT = w_hh.T
    bih = b_ih.reshape(1, 3 * D)
    bhh = b_hh.reshape(1, 3 * D)
    zeros = jnp.zeros((N, D), jnp.float32)

    h = x_encoded
    m = _mm(h, weight[0])
    for i in range(L):
        parts = _sc_segsum_call()(m, idx_main, idx_tail, zeros)
        if i + 1 < L:
            h, m = _gru_mm(parts[0], parts[1], h, wihT, whhT, bih, bhh,
                           weight[i + 1])
        else:
            h = _gru(parts[0], parts[1], h, wihT, whhT, bih, bhh)
    return h
